# baseline (speedup 1.0000x reference)
import functools

import jax
import jax.numpy as jnp
from jax import lax
from jax.experimental import pallas as pl
from jax.experimental.pallas import tpu as pltpu
from jax.experimental.pallas import tpu_sc as plsc

_L = 16
_CH = 128
_G = _CH // _L


def _pers_from_lanes(x, y, z, cam):
    r00, r01, r02, r10, r11, r12, r20, r21, r22, c0, c1, c2 = cam
    sx = x - c0
    sy = y - c1
    sz = z - c2
    u0 = r00 * sx + r10 * sy + r20 * sz
    u1 = r01 * sx + r11 * sy + r21 * sz
    u2 = r02 * sx + r12 * sy + r22 * sz
    return u0 / u2, u1 / u2, u2


def _fetch3(ab_ref, rows, o3):
    out = []
    for j in range(3):
        oj = o3 + j if j else o3
        row2 = rows + lax.shift_left(lax.shift_right_logical(oj, 5), 7)
        col2 = lax.bitwise_and(oj, 31)
        out.append(plsc.load_gather(ab_ref, [row2, col2]))
    return out


def _body(xyz_hbm, emb_hbm, conf_hbm, dir_hbm, color_hbm, idx_hbm, locw_hbm,
          cam_hbm, oemb_hbm, ocol_hbm, odir_hbm, oconf_hbm, oloc_hbm,
          idx_v, ga_v, gb_v, gc_v, o3_v, oc_v,
          xyz_ab, col_ab, dir_ab, conf_b, emb_v,
          oemb_v, ocol_v, odir_v, oconf_v, loc_v, oloc_v,
          cam_v, sem, osem):
    q_total = idx_hbm.shape[0]
    qloc_total = locw_hbm.shape[0] // 3
    gmax = xyz_hbm.shape[0] - 1
    nw = 32
    qw = q_total // nw
    qlw = qloc_total // nw
    wid = lax.axis_index("s") * 2 + lax.axis_index("c")

    pltpu.sync_copy(cam_hbm, cam_v)
    cam = tuple(cam_v[j, :] for j in range(12))
    iota = lax.iota(jnp.int32, _L)
    iota3 = iota * 3

    def chunk(it, carry):
        base = wid * qw + it * _CH
        pltpu.sync_copy(idx_hbm.at[pl.ds(base, _CH)], idx_v)
        for g in range(_G):
            sl = pl.ds(g * _L, _L)
            v = idx_v[sl]
            t = v * 3
            ga = lax.shift_right_logical(t, 5)
            ga_v[sl] = ga
            gb_v[sl] = jnp.minimum(ga + 1, gmax)
            gc_v[sl] = lax.shift_right_logical(v, 5)
            o3_v[sl] = lax.bitwise_and(t, 31)
            oc_v[sl] = lax.bitwise_and(v, 31)
        d = [
            pltpu.async_copy(emb_hbm.at[idx_v], emb_v, sem),
            pltpu.async_copy(xyz_hbm.at[ga_v], xyz_ab.at[pl.ds(0, _CH)], sem),
            pltpu.async_copy(xyz_hbm.at[gb_v], xyz_ab.at[pl.ds(_CH, _CH)], sem),
            pltpu.async_copy(color_hbm.at[ga_v], col_ab.at[pl.ds(0, _CH)], sem),
            pltpu.async_copy(color_hbm.at[gb_v], col_ab.at[pl.ds(_CH, _CH)], sem),
            pltpu.async_copy(dir_hbm.at[ga_v], dir_ab.at[pl.ds(0, _CH)], sem),
            pltpu.async_copy(dir_hbm.at[gb_v], dir_ab.at[pl.ds(_CH, _CH)], sem),
            pltpu.async_copy(conf_hbm.at[gc_v], conf_b, sem),
        ]
        for dd in d:
            dd.wait()
        for g in range(_G):
            sl = pl.ds(g * _L, _L)
            rows = iota + g * _L
            f38 = iota * 38 + g * (_L * 38)
            o3 = o3_v[sl]
            x, y, z = _fetch3(xyz_ab, rows, o3)
            xp, yp, zc = _pers_from_lanes(x, y, z, cam)
            for c, val in ((0, x), (1, y), (2, z), (3, xp), (4, yp), (5, zc)):
                plsc.store_scatter(oemb_v, [f38 + c], val)
            f3 = iota3 + g * (_L * 3)
            r, gg, b = _fetch3(col_ab, rows, o3)
            plsc.store_scatter(ocol_v, [f3], r)
            plsc.store_scatter(ocol_v, [f3 + 1], gg)
            plsc.store_scatter(ocol_v, [f3 + 2], b)
            dx, dy, dz = _fetch3(dir_ab, rows, o3)
            plsc.store_scatter(odir_v, [f3], dx)
            plsc.store_scatter(odir_v, [f3 + 1], dy)
            plsc.store_scatter(odir_v, [f3 + 2], dz)
            cf = plsc.load_gather(conf_b, [rows, oc_v[sl]])
            plsc.store_scatter(oconf_v, [rows], cf)
        for p in range(_CH):
            oemb_v[pl.ds(p * 38 + 6, 16)] = emb_v[p, pl.ds(0, 16)]
            oemb_v[pl.ds(p * 38 + 22, 16)] = emb_v[p, pl.ds(16, 16)]
        o = [
            pltpu.async_copy(oemb_v, oemb_hbm.at[pl.ds(base * 38, _CH * 38)],
                             osem),
            pltpu.async_copy(ocol_v, ocol_hbm.at[pl.ds(base * 3, _CH * 3)],
                             osem),
            pltpu.async_copy(odir_v, odir_hbm.at[pl.ds(base * 3, _CH * 3)],
                             osem),
            pltpu.async_copy(oconf_v, oconf_hbm.at[pl.ds(base, _CH)], osem),
        ]
        for oo in o:
            oo.wait()
        return carry

    lax.fori_loop(0, qw // _CH, chunk, 0)

    def loc_chunk(it, carry):
        base = wid * qlw + it * _CH
        pltpu.sync_copy(locw_hbm.at[pl.ds(base * 3, _CH * 3)], loc_v)
        for g in range(_G):
            f3 = iota3 + g * (_L * 3)
            x = plsc.load_gather(loc_v, [f3])
            y = plsc.load_gather(loc_v, [f3 + 1])
            z = plsc.load_gather(loc_v, [f3 + 2])
            xp, yp, zc = _pers_from_lanes(x, y, z, cam)
            plsc.store_scatter(oloc_v, [f3], xp)
            plsc.store_scatter(oloc_v, [f3 + 1], yp)
            plsc.store_scatter(oloc_v, [f3 + 2], zc)
        pltpu.sync_copy(oloc_v, oloc_hbm.at[pl.ds(base * 3, _CH * 3)])
        return carry

    lax.fori_loop(0, qlw // _CH, loc_chunk, 0)


def kernel(points_xyz, points_embeding, points_conf, points_dir, points_color,
           sample_pidx, sample_loc_w, cam_rot, cam_pos):
    n = points_xyz.shape[0]
    b, r, sr, k = sample_pidx.shape
    c = points_embeding.shape[-1]
    q = b * r * sr * k
    qloc = b * r * sr
    cf = c + 6

    idx = sample_pidx.reshape(q)
    emb2d = points_embeding.reshape(n, c)
    conf32 = points_conf.reshape(n // 32, 32)
    dir32 = points_dir.reshape(3 * n // 32, 32)
    color32 = points_color.reshape(3 * n // 32, 32)
    xyz32 = points_xyz.reshape(3 * n // 32, 32)
    locw = sample_loc_w.reshape(3 * qloc)
    cam12 = jnp.concatenate(
        [cam_rot.reshape(9), cam_pos.reshape(3)]).astype(jnp.float32)
    cam_b = jnp.tile(cam12[:, None], (1, _L))

    mesh = plsc.VectorSubcoreMesh(
        core_axis_name="c", subcore_axis_name="s", num_cores=2, num_subcores=16)
    f32 = jnp.float32
    i32 = jnp.int32
    out_type = (
        jax.ShapeDtypeStruct((q * cf,), f32),
        jax.ShapeDtypeStruct((q * 3,), f32),
        jax.ShapeDtypeStruct((q * 3,), f32),
        jax.ShapeDtypeStruct((q,), f32),
        jax.ShapeDtypeStruct((qloc * 3,), f32),
    )
    scratch = [
        pltpu.VMEM((_CH,), i32),
        pltpu.VMEM((_CH,), i32),
        pltpu.VMEM((_CH,), i32),
        pltpu.VMEM((_CH,), i32),
        pltpu.VMEM((_CH,), i32),
        pltpu.VMEM((_CH,), i32),
        pltpu.VMEM((2 * _CH, 32), f32),
        pltpu.VMEM((2 * _CH, 32), f32),
        pltpu.VMEM((2 * _CH, 32), f32),
        pltpu.VMEM((_CH, 32), f32),
        pltpu.VMEM((_CH, c), f32),
        pltpu.VMEM((_CH * cf,), f32),
        pltpu.VMEM((_CH * 3,), f32),
        pltpu.VMEM((_CH * 3,), f32),
        pltpu.VMEM((_CH,), f32),
        pltpu.VMEM((_CH * 3,), f32),
        pltpu.VMEM((_CH * 3,), f32),
        pltpu.VMEM((12, _L), f32),
        pltpu.SemaphoreType.DMA,
        pltpu.SemaphoreType.DMA,
    ]
    fn = pl.kernel(
        _body, out_type=out_type, mesh=mesh, scratch_types=scratch,
        compiler_params=pltpu.CompilerParams(
            needs_layout_passes=False, use_tc_tiling_on_sc=False))
    oemb, ocol, odir, oconf, oloc = fn(
        xyz32, emb2d, conf32, dir32, color32, idx, locw, cam_b)

    return (oemb.reshape(b, r, sr, k, cf),
            ocol.reshape(b, r, sr, k, 3),
            odir.reshape(b, r, sr, k, 3),
            oconf.reshape(b, r, sr, k, 1),
            oloc.reshape(b, r, sr, 3))


# speedup vs baseline: 4.2891x; 4.2891x over previous
import functools

import jax
import jax.numpy as jnp
from jax import lax
from jax.experimental import pallas as pl
from jax.experimental.pallas import tpu as pltpu
from jax.experimental.pallas import tpu_sc as plsc

_L = 16
_CH = 128
_G = _CH // _L


def _pers_from_lanes(x, y, z, cam):
    r00, r01, r02, r10, r11, r12, r20, r21, r22, c0, c1, c2 = cam
    sx = x - c0
    sy = y - c1
    sz = z - c2
    u0 = r00 * sx + r10 * sy + r20 * sz
    u1 = r01 * sx + r11 * sy + r21 * sz
    u2 = r02 * sx + r12 * sy + r22 * sz
    return u0 / u2, u1 / u2, u2


def _body(pk_hbm, emb_hbm, idx_hbm, locw_hbm, cam_hbm,
          oemb_hbm, ocol_hbm, odir_hbm, oconf_hbm, oloc_hbm,
          idx_v, pk_v, emb_v, oemb_v, ocol_v, odir_v, oconf_v,
          loc_v, oloc_v, cam_v, sem, osem):
    q_total = idx_hbm.shape[0] * idx_hbm.shape[1]
    qloc_total = locw_hbm.shape[0] * _CH
    nw = 32
    qw = q_total // nw
    qlw = qloc_total // nw
    wid = lax.axis_index("s") * 2 + lax.axis_index("c")

    pltpu.sync_copy(cam_hbm, cam_v)
    cam = tuple(cam_v[j, :] for j in range(12))
    iota = lax.iota(jnp.int32, _L)
    iota3 = iota * 3
    cols = [jnp.full((_L,), c, jnp.int32) for c in range(10)]

    def chunk(it, carry):
        base = wid * qw + it * _CH
        pltpu.sync_copy(idx_hbm.at[base // _CH], idx_v)
        d0 = pltpu.async_copy(emb_hbm.at[idx_v], emb_v, sem)
        d1 = pltpu.async_copy(pk_hbm.at[idx_v], pk_v, sem)
        d0.wait()
        d1.wait()
        for g in range(_G):
            rows = iota + g * _L
            f38 = iota * 38 + g * (_L * 38)
            f3 = iota3 + g * (_L * 3)
            x = plsc.load_gather(pk_v, [rows, cols[0]])
            y = plsc.load_gather(pk_v, [rows, cols[1]])
            z = plsc.load_gather(pk_v, [rows, cols[2]])
            xp, yp, zc = _pers_from_lanes(x, y, z, cam)
            for c, val in ((0, x), (1, y), (2, z), (3, xp), (4, yp), (5, zc)):
                plsc.store_scatter(oemb_v, [f38 + c], val)
            for c in range(3):
                plsc.store_scatter(
                    ocol_v, [f3 + c], plsc.load_gather(pk_v, [rows, cols[3 + c]]))
            for c in range(3):
                plsc.store_scatter(
                    odir_v, [f3 + c], plsc.load_gather(pk_v, [rows, cols[6 + c]]))
            plsc.store_scatter(
                oconf_v, [rows], plsc.load_gather(pk_v, [rows, cols[9]]))
        for p in range(_CH):
            oemb_v[pl.ds(p * 38 + 6, 16)] = emb_v[p, pl.ds(0, 16)]
            oemb_v[pl.ds(p * 38 + 22, 16)] = emb_v[p, pl.ds(16, 16)]
        o = [
            pltpu.async_copy(oemb_v, oemb_hbm.at[pl.ds(base * 38, _CH * 38)],
                             osem),
            pltpu.async_copy(ocol_v, ocol_hbm.at[pl.ds(base * 3, _CH * 3)],
                             osem),
            pltpu.async_copy(odir_v, odir_hbm.at[pl.ds(base * 3, _CH * 3)],
                             osem),
            pltpu.async_copy(oconf_v, oconf_hbm.at[pl.ds(base, _CH)], osem),
        ]
        for oo in o:
            oo.wait()
        return carry

    lax.fori_loop(0, qw // _CH, chunk, 0)

    def loc_chunk(it, carry):
        base = wid * qlw + it * _CH
        pltpu.sync_copy(locw_hbm.at[base // _CH], loc_v)
        for g in range(_G):
            f3 = iota3 + g * (_L * 3)
            x = plsc.load_gather(loc_v, [f3])
            y = plsc.load_gather(loc_v, [f3 + 1])
            z = plsc.load_gather(loc_v, [f3 + 2])
            xp, yp, zc = _pers_from_lanes(x, y, z, cam)
            plsc.store_scatter(oloc_v, [f3], xp)
            plsc.store_scatter(oloc_v, [f3 + 1], yp)
            plsc.store_scatter(oloc_v, [f3 + 2], zc)
        pltpu.sync_copy(oloc_v, oloc_hbm.at[base // _CH])
        return carry

    lax.fori_loop(0, qlw // _CH, loc_chunk, 0)


def kernel(points_xyz, points_embeding, points_conf, points_dir, points_color,
           sample_pidx, sample_loc_w, cam_rot, cam_pos):
    n = points_xyz.shape[0]
    b, r, sr, k = sample_pidx.shape
    c = points_embeding.shape[-1]
    q = b * r * sr * k
    qloc = b * r * sr
    cf = c + 6

    packed = jnp.concatenate(
        [points_xyz, points_color[0], points_dir[0], points_conf[0],
         jnp.zeros((n, 6), jnp.float32)], axis=-1)
    emb2d = points_embeding.reshape(n, c)
    idx2d = sample_pidx.reshape(q // _CH, _CH)
    locw2d = sample_loc_w.reshape(qloc // _CH, 3 * _CH)
    cam12 = jnp.concatenate(
        [cam_rot.reshape(9), cam_pos.reshape(3)]).astype(jnp.float32)
    cam_b = jnp.tile(cam12[:, None], (1, _L))

    mesh = plsc.VectorSubcoreMesh(
        core_axis_name="c", subcore_axis_name="s", num_cores=2, num_subcores=16)
    f32 = jnp.float32
    i32 = jnp.int32
    out_type = (
        jax.ShapeDtypeStruct((q * cf,), f32),
        jax.ShapeDtypeStruct((q * 3,), f32),
        jax.ShapeDtypeStruct((q * 3,), f32),
        jax.ShapeDtypeStruct((q,), f32),
        jax.ShapeDtypeStruct((qloc // _CH, 3 * _CH), f32),
    )
    scratch = [
        pltpu.VMEM((_CH,), i32),
        pltpu.VMEM((_CH, 16), f32),
        pltpu.VMEM((_CH, c), f32),
        pltpu.VMEM((_CH * cf,), f32),
        pltpu.VMEM((_CH * 3,), f32),
        pltpu.VMEM((_CH * 3,), f32),
        pltpu.VMEM((_CH,), f32),
        pltpu.VMEM((_CH * 3,), f32),
        pltpu.VMEM((_CH * 3,), f32),
        pltpu.VMEM((12, _L), f32),
        pltpu.SemaphoreType.DMA,
        pltpu.SemaphoreType.DMA,
    ]
    fn = pl.kernel(
        _body, out_type=out_type, mesh=mesh, scratch_types=scratch,
        compiler_params=pltpu.CompilerParams(
            needs_layout_passes=False, use_tc_tiling_on_sc=False))
    oemb, ocol, odir, oconf, oloc = fn(packed, emb2d, idx2d, locw2d, cam_b)

    return (oemb.reshape(b, r, sr, k, cf),
            ocol.reshape(b, r, sr, k, 3),
            odir.reshape(b, r, sr, k, 3),
            oconf.reshape(b, r, sr, k, 1),
            oloc.reshape(b, r, sr, 3))
